# Initial kernel scaffold; baseline (speedup 1.0000x reference)
import jax
import jax.numpy as jnp
from jax import lax
from jax.experimental import pallas as pl
from jax.experimental.pallas import tpu as pltpu
from jax.experimental.pallas import tpu_sc as plsc

f32 = jnp.float32
i32 = jnp.int32

N = 10000
E = 320000
H = 128
NC = 2
NS = 16
NW = NC * NS
CHUNK = 128
NCHUNKS = E // CHUNK
NPT = N // NS
CPW = NCHUNKS // NW
CPW_REM = NCHUNKS - CPW * NW

_SC_PARAMS = pltpu.CompilerParams(needs_layout_passes=False)


def _silu(v):
    return v * jax.nn.sigmoid(v)


def _worker_chunk_range(w):
    start = w * CPW + jnp.minimum(w, CPW_REM)
    count = jnp.where(w < CPW_REM, CPW + 1, CPW)
    return start, count


def _sc_mesh():
    return plsc.VectorSubcoreMesh(
        core_axis_name="c", subcore_axis_name="s",
        num_cores=NC, num_subcores=NS)


def _sc_d2_cnt_body(row_h, col_h, px_h, py_h, pz_h, d2_h, cntp_h,
                    pxv, pyv, pzv, rowv, colv, d2v, cntv):
    w = lax.axis_index("c") * NS + lax.axis_index("s")
    pltpu.sync_copy(px_h, pxv)
    pltpu.sync_copy(py_h, pyv)
    pltpu.sync_copy(pz_h, pzv)

    def zero_cnt(i, _):
        cntv[pl.ds(i * 16, 16)] = jnp.zeros((16,), f32)
        return 0
    lax.fori_loop(0, N // 16, zero_cnt, 0)

    start, count = _worker_chunk_range(w)
    ones16 = jnp.ones((16,), f32)

    def chunk(j, _):
        off = (start + j) * CHUNK
        pltpu.sync_copy(row_h.at[pl.ds(off, CHUNK)], rowv)
        pltpu.sync_copy(col_h.at[pl.ds(off, CHUNK)], colv)

        def grp(gi, _):
            sl = pl.ds(gi * 16, 16)
            r = rowv[sl]
            c = colv[sl]
            dx = plsc.load_gather(pxv, [r]) - plsc.load_gather(pxv, [c])
            dy = plsc.load_gather(pyv, [r]) - plsc.load_gather(pyv, [c])
            dz = plsc.load_gather(pzv, [r]) - plsc.load_gather(pzv, [c])
            d2v[sl] = dx * dx + dy * dy + dz * dz
            plsc.addupdate_scatter(cntv, [c], ones16)
            return 0
        lax.fori_loop(0, CHUNK // 16, grp, 0)
        pltpu.sync_copy(d2v, d2_h.at[pl.ds(off, CHUNK)])
        return 0
    lax.fori_loop(0, count, chunk, 0)
    pltpu.sync_copy(cntv, cntp_h.at[w])


def _sc_d2_cnt(row, col, px, py, pz):
    fn = pl.kernel(
        _sc_d2_cnt_body,
        out_type=(jax.ShapeDtypeStruct((E,), f32),
                  jax.ShapeDtypeStruct((NW, N), f32)),
        mesh=_sc_mesh(),
        compiler_params=_SC_PARAMS,
        scratch_types=[
            pltpu.VMEM((N,), f32), pltpu.VMEM((N,), f32), pltpu.VMEM((N,), f32),
            pltpu.VMEM((CHUNK,), i32), pltpu.VMEM((CHUNK,), i32),
            pltpu.VMEM((CHUNK,), f32), pltpu.VMEM((N,), f32),
        ],
    )
    return fn(row, col, px, py, pz)


def _sc_edge_body(a_h, b_h, row_h, col_h, dist_h, w1c_h, accp_h,
                  acc, abuf, bbuf, rowv, colv, distv, w1cv, sem_a, sem_b):
    c = lax.axis_index("c")
    s = lax.axis_index("s")
    w = c * NS + s
    pltpu.sync_copy(w1c_h, w1cv)

    def zero_row(i, _):
        for k in range(H // 16):
            abuf[i, pl.ds(16 * k, 16)] = jnp.zeros((16,), f32)
        return 0
    lax.fori_loop(0, 125, zero_row, 0)
    for t in range(NPT // 125):
        pltpu.sync_copy(abuf.at[pl.ds(0, 125)],
                        acc.at[pl.ds(s * NPT + t * 125, 125)])
    plsc.subcore_barrier()

    start, count = _worker_chunk_range(w)

    def chunk(j, _):
        off = (start + j) * CHUNK
        pltpu.sync_copy(row_h.at[pl.ds(off, CHUNK)], rowv)
        pltpu.sync_copy(col_h.at[pl.ds(off, CHUNK)], colv)
        pltpu.sync_copy(dist_h.at[pl.ds(off, CHUNK)], distv)
        cp_a = pltpu.async_copy(a_h.at[colv], abuf, sem_a)
        cp_b = pltpu.async_copy(b_h.at[rowv], bbuf, sem_b)
        cp_a.wait()
        cp_b.wait()

        def erow(e, _):
            d16 = plsc.load_gather(distv, [jnp.full((16,), e, dtype=i32)])
            for k in range(H // 16):
                sl = pl.ds(16 * k, 16)
                v = abuf[e, sl] + bbuf[e, sl] + d16 * w1cv[sl]
                sg = 1.0 / (1.0 + jnp.exp(-v))
                abuf[e, sl] = v * sg
            return 0
        lax.fori_loop(0, CHUNK, erow, 0)
        pltpu.sync_copy(abuf, acc.at[colv], add=True)
        return 0
    lax.fori_loop(0, count, chunk, 0)
    plsc.subcore_barrier()
    pltpu.sync_copy(acc.at[pl.ds(s * NPT, NPT)],
                    accp_h.at[c, pl.ds(s * NPT, NPT)])


def _sc_edge(a_tab, b_tab, row, col, dist, w1c):
    fn = pl.kernel(
        _sc_edge_body,
        out_type=jax.ShapeDtypeStruct((NC, N, H), f32),
        mesh=_sc_mesh(),
        compiler_params=_SC_PARAMS,
        scratch_types=[
            pltpu.VMEM_SHARED((N, H), f32),
            pltpu.VMEM((CHUNK, H), f32), pltpu.VMEM((CHUNK, H), f32),
            pltpu.VMEM((CHUNK,), i32), pltpu.VMEM((CHUNK,), i32),
            pltpu.VMEM((CHUNK,), f32), pltpu.VMEM((H,), f32),
            pltpu.SemaphoreType.DMA, pltpu.SemaphoreType.DMA,
        ],
    )
    return fn(a_tab, b_tab, row, col, dist, w1c)


def _tc_pre_body(x_r, temb_r, tw1t_r, tb1_r, tw2t_r, tb2_r,
                 w1at_r, b1_r, w1bt_r, h_o, a_o, b_o):
    t = _silu(jnp.dot(temb_r[...], tw1t_r[...], preferred_element_type=f32)
              + tb1_r[...])
    tp = jnp.dot(t, tw2t_r[...], preferred_element_type=f32) + tb2_r[...]
    h = x_r[...] + tp
    h_o[...] = h
    a_o[...] = jnp.dot(h, w1at_r[...], preferred_element_type=f32) + b1_r[...]
    b_o[...] = jnp.dot(h, w1bt_r[...], preferred_element_type=f32)


def _tc_pre(x, time_emb, tw1t, tb1, tw2t, tb2, w1at, b1, w1bt):
    return pl.pallas_call(
        _tc_pre_body,
        out_shape=(jax.ShapeDtypeStruct((N, H), f32),
                   jax.ShapeDtypeStruct((N, H), f32),
                   jax.ShapeDtypeStruct((N, H), f32)),
    )(x, time_emb, tw1t, tb1, tw2t, tb2, w1at, b1, w1bt)


def _tc_dist_body(d2_r, cntp_r, dist_o, inv_o):
    dist_o[...] = jnp.sqrt(d2_r[...] + 1e-12)
    cnt = jnp.sum(cntp_r[...], axis=0, keepdims=True)
    inv_o[...] = 1.0 / jnp.maximum(cnt, 1.0)


def _tc_dist(d2m, cntp):
    return pl.pallas_call(
        _tc_dist_body,
        out_shape=(jax.ShapeDtypeStruct(d2m.shape, f32),
                   jax.ShapeDtypeStruct((1, N), f32)),
    )(d2m, cntp)


def _layer_core(h_r, accp_r, inv_r, ew2t_r, eb2_r, nw1t_r, nb1_r,
                nw2t_r, nb2_r, g_r, b_r):
    h = h_r[...]
    acc = (accp_r[0] + accp_r[1]) * inv_r[...]
    agg = jnp.dot(acc, ew2t_r[...], preferred_element_type=f32) + eb2_r[...]
    t = _silu(jnp.dot(h, nw1t_r[...], preferred_element_type=f32) + nb1_r[...])
    nm = jnp.dot(t, nw2t_r[...], preferred_element_type=f32) + nb2_r[...]
    y = h + nm + agg
    mu = jnp.mean(y, axis=-1, keepdims=True)
    yc = y - mu
    var = jnp.mean(yc * yc, axis=-1, keepdims=True)
    return yc * lax.rsqrt(var + 1e-5) * g_r[...] + b_r[...]


def _tc_layer_ab_body(h_r, accp_r, inv_r, ew2t_r, eb2_r, nw1t_r, nb1_r,
                      nw2t_r, nb2_r, g_r, b_r, w1at_r, b1_r, w1bt_r,
                      h_o, a_o, b_o):
    hn = _layer_core(h_r, accp_r, inv_r, ew2t_r, eb2_r, nw1t_r, nb1_r,
                     nw2t_r, nb2_r, g_r, b_r)
    h_o[...] = hn
    a_o[...] = jnp.dot(hn, w1at_r[...], preferred_element_type=f32) + b1_r[...]
    b_o[...] = jnp.dot(hn, w1bt_r[...], preferred_element_type=f32)


def _tc_layer_final_body(h_r, accp_r, inv_r, ew2t_r, eb2_r, nw1t_r, nb1_r,
                         nw2t_r, nb2_r, g_r, b_r, h_o):
    h_o[...] = _layer_core(h_r, accp_r, inv_r, ew2t_r, eb2_r, nw1t_r, nb1_r,
                           nw2t_r, nb2_r, g_r, b_r)


def _tc_layer_ab(h, accp, inv, ew2t, eb2, nw1t, nb1, nw2t, nb2, g, b,
                 w1at_n, b1_n, w1bt_n):
    return pl.pallas_call(
        _tc_layer_ab_body,
        out_shape=(jax.ShapeDtypeStruct((N, H), f32),
                   jax.ShapeDtypeStruct((N, H), f32),
                   jax.ShapeDtypeStruct((N, H), f32)),
    )(h, accp, inv, ew2t, eb2, nw1t, nb1, nw2t, nb2, g, b,
      w1at_n, b1_n, w1bt_n)


def _tc_layer_final(h, accp, inv, ew2t, eb2, nw1t, nb1, nw2t, nb2, g, b):
    return pl.pallas_call(
        _tc_layer_final_body,
        out_shape=jax.ShapeDtypeStruct((N, H), f32),
    )(h, accp, inv, ew2t, eb2, nw1t, nb1, nw2t, nb2, g, b)


def kernel(x, pos, edge_index, time_emb, t_w1, t_b1, t_w2, t_b2,
           edge_w1, edge_b1, edge_w2, edge_b2, node_w1, node_b1,
           node_w2, node_b2, ln_g, ln_b):
    row = edge_index[0]
    col = edge_index[1]
    px = pos[:, 0]
    py = pos[:, 1]
    pz = pos[:, 2]

    w1at = [edge_w1[l][:, :H].T for l in range(3)]
    w1bt = [edge_w1[l][:, H:2 * H].T for l in range(3)]
    w1c = [edge_w1[l][:, 2 * H] for l in range(3)]
    b1 = [edge_b1[l][None, :] for l in range(3)]
    ew2t = [edge_w2[l].T for l in range(3)]
    eb2 = [edge_b2[l][None, :] for l in range(3)]
    nw1t = [node_w1[l].T for l in range(3)]
    nb1 = [node_b1[l][None, :] for l in range(3)]
    nw2t = [node_w2[l].T for l in range(3)]
    nb2 = [node_b2[l][None, :] for l in range(3)]
    g = [ln_g[l][None, :] for l in range(3)]
    b = [ln_b[l][None, :] for l in range(3)]

    h, a_tab, b_tab = _tc_pre(x, time_emb, t_w1.T, t_b1[None, :], t_w2.T,
                              t_b2[None, :], w1at[0], b1[0], w1bt[0])
    d2, cntp = _sc_d2_cnt(row, col, px, py, pz)
    dist2d, inv1n = _tc_dist(d2.reshape(NCHUNKS, CHUNK), cntp)
    dist = dist2d.reshape(E)
    inv = inv1n.reshape(N, 1)

    for l in range(3):
        accp = _sc_edge(a_tab, b_tab, row, col, dist, w1c[l])
        if l < 2:
            h, a_tab, b_tab = _tc_layer_ab(
                h, accp, inv, ew2t[l], eb2[l], nw1t[l], nb1[l], nw2t[l],
                nb2[l], g[l], b[l], w1at[l + 1], b1[l + 1], w1bt[l + 1])
        else:
            h = _tc_layer_final(
                h, accp, inv, ew2t[l], eb2[l], nw1t[l], nb1[l], nw2t[l],
                nb2[l], g[l], b[l])
    return h, pos


# speedup vs baseline: 1.3732x; 1.3732x over previous
import jax
import jax.numpy as jnp
from jax import lax
from jax.experimental import pallas as pl
from jax.experimental.pallas import tpu as pltpu
from jax.experimental.pallas import tpu_sc as plsc

f32 = jnp.float32
i32 = jnp.int32

N = 10000
E = 320000
H = 128
NC = 2
NS = 16
NW = NC * NS
CHUNK = 128
NCHUNKS = E // CHUNK
NPT = 624
NTAIL = N - NPT * NS
CPW = NCHUNKS // NW
CPW_REM = NCHUNKS - CPW * NW

_SC_PARAMS = pltpu.CompilerParams(needs_layout_passes=False)


def _silu(v):
    return v * jax.nn.sigmoid(v)


def _worker_chunk_range(w):
    start = w * CPW + jnp.minimum(w, CPW_REM)
    count = jnp.where(w < CPW_REM, CPW + 1, CPW)
    return start, count


def _sc_mesh():
    return plsc.VectorSubcoreMesh(
        core_axis_name="c", subcore_axis_name="s",
        num_cores=NC, num_subcores=NS)


def _sc_d2_cnt_body(row_h, col_h, px_h, py_h, pz_h, d2_h, cntp_h,
                    pxv, pyv, pzv, rowv, colv, d2v, cntv):
    w = lax.axis_index("c") * NS + lax.axis_index("s")
    pltpu.sync_copy(px_h, pxv)
    pltpu.sync_copy(py_h, pyv)
    pltpu.sync_copy(pz_h, pzv)

    def zero_cnt(i, _):
        cntv[pl.ds(i * 16, 16)] = jnp.zeros((16,), f32)
        return 0
    lax.fori_loop(0, N // 16, zero_cnt, 0)

    start, count = _worker_chunk_range(w)
    ones16 = jnp.ones((16,), f32)

    def chunk(j, _):
        off = (start + j) * CHUNK
        pltpu.sync_copy(row_h.at[pl.ds(off, CHUNK)], rowv)
        pltpu.sync_copy(col_h.at[pl.ds(off, CHUNK)], colv)

        def grp(gi, _):
            sl = pl.ds(gi * 16, 16)
            r = rowv[sl]
            c = colv[sl]
            dx = plsc.load_gather(pxv, [r]) - plsc.load_gather(pxv, [c])
            dy = plsc.load_gather(pyv, [r]) - plsc.load_gather(pyv, [c])
            dz = plsc.load_gather(pzv, [r]) - plsc.load_gather(pzv, [c])
            d2v[sl] = dx * dx + dy * dy + dz * dz
            plsc.addupdate_scatter(cntv, [c], ones16)
            return 0
        lax.fori_loop(0, CHUNK // 16, grp, 0)
        pltpu.sync_copy(d2v, d2_h.at[pl.ds(off, CHUNK)])
        return 0
    lax.fori_loop(0, count, chunk, 0)
    pltpu.sync_copy(cntv, cntp_h.at[pl.ds(w * N, N)])


def _sc_d2_cnt(row, col, px, py, pz):
    fn = pl.kernel(
        _sc_d2_cnt_body,
        out_type=(jax.ShapeDtypeStruct((E,), f32),
                  jax.ShapeDtypeStruct((NW * N,), f32)),
        mesh=_sc_mesh(),
        compiler_params=_SC_PARAMS,
        scratch_types=[
            pltpu.VMEM((N,), f32), pltpu.VMEM((N,), f32), pltpu.VMEM((N,), f32),
            pltpu.VMEM((CHUNK,), i32), pltpu.VMEM((CHUNK,), i32),
            pltpu.VMEM((CHUNK,), f32), pltpu.VMEM((N,), f32),
        ],
    )
    return fn(row, col, px, py, pz)


def _sc_edge_body(a_h, b_h, row_h, col_h, dist_h, w1c_h, accp_h,
                  acc, abuf, bbuf, rowv, colv, distv, w1cv, sem_a, sem_b):
    c = lax.axis_index("c")
    s = lax.axis_index("s")
    w = c * NS + s
    pltpu.sync_copy(w1c_h, w1cv)

    def zero_row(i, _):
        for k in range(H // 16):
            abuf[i, pl.ds(16 * k, 16)] = jnp.zeros((16,), f32)
        return 0
    lax.fori_loop(0, 104, zero_row, 0)
    for t in range(NPT // 104):
        pltpu.sync_copy(abuf.at[pl.ds(0, 104)],
                        acc.at[pl.ds(s * NPT + t * 104, 104)])

    @pl.when(s == NS - 1)
    def _():
        pltpu.sync_copy(abuf.at[pl.ds(0, NTAIL)],
                        acc.at[pl.ds(NPT * NS, NTAIL)])
    plsc.subcore_barrier()

    start, count = _worker_chunk_range(w)

    def chunk(j, _):
        off = (start + j) * CHUNK
        pltpu.sync_copy(row_h.at[pl.ds(off, CHUNK)], rowv)
        pltpu.sync_copy(col_h.at[pl.ds(off, CHUNK)], colv)
        pltpu.sync_copy(dist_h.at[pl.ds(off, CHUNK)], distv)
        cp_a = pltpu.async_copy(a_h.at[colv], abuf, sem_a)
        cp_b = pltpu.async_copy(b_h.at[rowv], bbuf, sem_b)
        cp_a.wait()
        cp_b.wait()

        def erow(e, _):
            d16 = plsc.load_gather(distv, [jnp.full((16,), e, dtype=i32)])
            for k in range(H // 16):
                sl = pl.ds(16 * k, 16)
                v = abuf[e, sl] + bbuf[e, sl] + d16 * w1cv[sl]
                sg = 1.0 / (1.0 + jnp.exp(-v))
                abuf[e, sl] = v * sg
            return 0
        lax.fori_loop(0, CHUNK, erow, 0)
        pltpu.sync_copy(abuf, acc.at[colv], add=True)
        return 0
    lax.fori_loop(0, count, chunk, 0)
    plsc.subcore_barrier()
    pltpu.sync_copy(acc.at[pl.ds(s * NPT, NPT)],
                    accp_h.at[c, pl.ds(s * NPT, NPT)])

    @pl.when(s == NS - 1)
    def _():
        pltpu.sync_copy(acc.at[pl.ds(NPT * NS, NTAIL)],
                        accp_h.at[c, pl.ds(NPT * NS, NTAIL)])


def _sc_edge(a_tab, b_tab, row, col, dist, w1c):
    fn = pl.kernel(
        _sc_edge_body,
        out_type=jax.ShapeDtypeStruct((NC, N, H), f32),
        mesh=_sc_mesh(),
        compiler_params=_SC_PARAMS,
        scratch_types=[
            pltpu.VMEM_SHARED((N, H), f32),
            pltpu.VMEM((CHUNK, H), f32), pltpu.VMEM((CHUNK, H), f32),
            pltpu.VMEM((CHUNK,), i32), pltpu.VMEM((CHUNK,), i32),
            pltpu.VMEM((CHUNK,), f32), pltpu.VMEM((H,), f32),
            pltpu.SemaphoreType.DMA, pltpu.SemaphoreType.DMA,
        ],
    )
    return fn(a_tab, b_tab, row, col, dist, w1c)


def _tc_pre_body(x_r, temb_r, tw1t_r, tb1_r, tw2t_r, tb2_r,
                 w1at_r, b1_r, w1bt_r, h_o, a_o, b_o):
    t = _silu(jnp.dot(temb_r[...], tw1t_r[...], preferred_element_type=f32)
              + tb1_r[...])
    tp = jnp.dot(t, tw2t_r[...], preferred_element_type=f32) + tb2_r[...]
    h = x_r[...] + tp
    h_o[...] = h
    a_o[...] = jnp.dot(h, w1at_r[...], preferred_element_type=f32) + b1_r[...]
    b_o[...] = jnp.dot(h, w1bt_r[...], preferred_element_type=f32)


def _tc_pre(x, time_emb, tw1t, tb1, tw2t, tb2, w1at, b1, w1bt):
    return pl.pallas_call(
        _tc_pre_body,
        out_shape=(jax.ShapeDtypeStruct((N, H), f32),
                   jax.ShapeDtypeStruct((N, H), f32),
                   jax.ShapeDtypeStruct((N, H), f32)),
    )(x, time_emb, tw1t, tb1, tw2t, tb2, w1at, b1, w1bt)


def _tc_dist_body(d2_r, cntp_r, dist_o, inv_o):
    dist_o[...] = jnp.sqrt(d2_r[...] + 1e-12)
    cnt = jnp.sum(cntp_r[...], axis=0, keepdims=True)
    inv_o[...] = 1.0 / jnp.maximum(cnt, 1.0)


def _tc_dist(d2m, cntp):
    return pl.pallas_call(
        _tc_dist_body,
        out_shape=(jax.ShapeDtypeStruct(d2m.shape, f32),
                   jax.ShapeDtypeStruct((1, N), f32)),
    )(d2m, cntp)


def _layer_core(h_r, accp_r, inv_r, ew2t_r, eb2_r, nw1t_r, nb1_r,
                nw2t_r, nb2_r, g_r, b_r):
    h = h_r[...]
    acc = (accp_r[0] + accp_r[1]) * inv_r[...]
    agg = jnp.dot(acc, ew2t_r[...], preferred_element_type=f32) + eb2_r[...]
    t = _silu(jnp.dot(h, nw1t_r[...], preferred_element_type=f32) + nb1_r[...])
    nm = jnp.dot(t, nw2t_r[...], preferred_element_type=f32) + nb2_r[...]
    y = h + nm + agg
    mu = jnp.mean(y, axis=-1, keepdims=True)
    yc = y - mu
    var = jnp.mean(yc * yc, axis=-1, keepdims=True)
    return yc * lax.rsqrt(var + 1e-5) * g_r[...] + b_r[...]


def _tc_layer_ab_body(h_r, accp_r, inv_r, ew2t_r, eb2_r, nw1t_r, nb1_r,
                      nw2t_r, nb2_r, g_r, b_r, w1at_r, b1_r, w1bt_r,
                      h_o, a_o, b_o):
    hn = _layer_core(h_r, accp_r, inv_r, ew2t_r, eb2_r, nw1t_r, nb1_r,
                     nw2t_r, nb2_r, g_r, b_r)
    h_o[...] = hn
    a_o[...] = jnp.dot(hn, w1at_r[...], preferred_element_type=f32) + b1_r[...]
    b_o[...] = jnp.dot(hn, w1bt_r[...], preferred_element_type=f32)


def _tc_layer_final_body(h_r, accp_r, inv_r, ew2t_r, eb2_r, nw1t_r, nb1_r,
                         nw2t_r, nb2_r, g_r, b_r, h_o):
    h_o[...] = _layer_core(h_r, accp_r, inv_r, ew2t_r, eb2_r, nw1t_r, nb1_r,
                           nw2t_r, nb2_r, g_r, b_r)


def _tc_layer_ab(h, accp, inv, ew2t, eb2, nw1t, nb1, nw2t, nb2, g, b,
                 w1at_n, b1_n, w1bt_n):
    return pl.pallas_call(
        _tc_layer_ab_body,
        out_shape=(jax.ShapeDtypeStruct((N, H), f32),
                   jax.ShapeDtypeStruct((N, H), f32),
                   jax.ShapeDtypeStruct((N, H), f32)),
    )(h, accp, inv, ew2t, eb2, nw1t, nb1, nw2t, nb2, g, b,
      w1at_n, b1_n, w1bt_n)


def _tc_layer_final(h, accp, inv, ew2t, eb2, nw1t, nb1, nw2t, nb2, g, b):
    return pl.pallas_call(
        _tc_layer_final_body,
        out_shape=jax.ShapeDtypeStruct((N, H), f32),
    )(h, accp, inv, ew2t, eb2, nw1t, nb1, nw2t, nb2, g, b)


def kernel(x, pos, edge_index, time_emb, t_w1, t_b1, t_w2, t_b2,
           edge_w1, edge_b1, edge_w2, edge_b2, node_w1, node_b1,
           node_w2, node_b2, ln_g, ln_b):
    row = edge_index[0]
    col = edge_index[1]
    px = pos[:, 0]
    py = pos[:, 1]
    pz = pos[:, 2]

    w1at = [edge_w1[l][:, :H].T for l in range(3)]
    w1bt = [edge_w1[l][:, H:2 * H].T for l in range(3)]
    w1c = [edge_w1[l][:, 2 * H] for l in range(3)]
    b1 = [edge_b1[l][None, :] for l in range(3)]
    ew2t = [edge_w2[l].T for l in range(3)]
    eb2 = [edge_b2[l][None, :] for l in range(3)]
    nw1t = [node_w1[l].T for l in range(3)]
    nb1 = [node_b1[l][None, :] for l in range(3)]
    nw2t = [node_w2[l].T for l in range(3)]
    nb2 = [node_b2[l][None, :] for l in range(3)]
    g = [ln_g[l][None, :] for l in range(3)]
    b = [ln_b[l][None, :] for l in range(3)]

    h, a_tab, b_tab = _tc_pre(x, time_emb, t_w1.T, t_b1[None, :], t_w2.T,
                              t_b2[None, :], w1at[0], b1[0], w1bt[0])
    d2, cntp = _sc_d2_cnt(row, col, px, py, pz)
    dist2d, inv1n = _tc_dist(d2.reshape(NCHUNKS, CHUNK),
                             cntp.reshape(NW, N))
    dist = dist2d.reshape(E)
    inv = inv1n.reshape(N, 1)

    for l in range(3):
        accp = _sc_edge(a_tab, b_tab, row, col, dist, w1c[l])
        if l < 2:
            h, a_tab, b_tab = _tc_layer_ab(
                h, accp, inv, ew2t[l], eb2[l], nw1t[l], nb1[l], nw2t[l],
                nb2[l], g[l], b[l], w1at[l + 1], b1[l + 1], w1bt[l + 1])
        else:
            h = _tc_layer_final(
                h, accp, inv, ew2t[l], eb2[l], nw1t[l], nb1[l], nw2t[l],
                nb2[l], g[l], b[l])
    return h, pos


# speedup vs baseline: 1.6055x; 1.1691x over previous
import jax
import jax.numpy as jnp
from jax import lax
from jax.experimental import pallas as pl
from jax.experimental.pallas import tpu as pltpu
from jax.experimental.pallas import tpu_sc as plsc

f32 = jnp.float32
i32 = jnp.int32

N = 10000
E = 320000
H = 128
NC = 2
NS = 16
NW = NC * NS
CHUNK = 128
NCHUNKS = E // CHUNK
NPT = 624
NTAIL = N - NPT * NS
CPW = NCHUNKS // NW
CPW_REM = NCHUNKS - CPW * NW

_SC_PARAMS = pltpu.CompilerParams(needs_layout_passes=False)


def _silu(v):
    return v * jax.nn.sigmoid(v)


def _worker_chunk_range(w):
    start = w * CPW + jnp.minimum(w, CPW_REM)
    count = jnp.where(w < CPW_REM, CPW + 1, CPW)
    return start, count


def _sc_mesh():
    return plsc.VectorSubcoreMesh(
        core_axis_name="c", subcore_axis_name="s",
        num_cores=NC, num_subcores=NS)


def _sc_d2_cnt_body(row_h, col_h, px_h, py_h, pz_h, d2_h, cntp_h,
                    pxv, pyv, pzv, rowv, colv, d2v, cntv):
    w = lax.axis_index("c") * NS + lax.axis_index("s")
    pltpu.sync_copy(px_h, pxv)
    pltpu.sync_copy(py_h, pyv)
    pltpu.sync_copy(pz_h, pzv)

    def zero_cnt(i, _):
        cntv[pl.ds(i * 16, 16)] = jnp.zeros((16,), f32)
        return 0
    lax.fori_loop(0, N // 16, zero_cnt, 0)

    start, count = _worker_chunk_range(w)
    ones16 = jnp.ones((16,), f32)

    def chunk(j, _):
        off = (start + j) * CHUNK
        pltpu.sync_copy(row_h.at[pl.ds(off, CHUNK)], rowv)
        pltpu.sync_copy(col_h.at[pl.ds(off, CHUNK)], colv)

        def grp(gi, _):
            sl = pl.ds(gi * 16, 16)
            r = rowv[sl]
            c = colv[sl]
            dx = plsc.load_gather(pxv, [r]) - plsc.load_gather(pxv, [c])
            dy = plsc.load_gather(pyv, [r]) - plsc.load_gather(pyv, [c])
            dz = plsc.load_gather(pzv, [r]) - plsc.load_gather(pzv, [c])
            d2v[sl] = dx * dx + dy * dy + dz * dz
            plsc.addupdate_scatter(cntv, [c], ones16)
            return 0
        lax.fori_loop(0, CHUNK // 16, grp, 0)
        pltpu.sync_copy(d2v, d2_h.at[pl.ds(off, CHUNK)])
        return 0
    lax.fori_loop(0, count, chunk, 0)
    pltpu.sync_copy(cntv, cntp_h.at[pl.ds(w * N, N)])


def _sc_d2_cnt(row, col, px, py, pz):
    fn = pl.kernel(
        _sc_d2_cnt_body,
        out_type=(jax.ShapeDtypeStruct((E,), f32),
                  jax.ShapeDtypeStruct((NW * N,), f32)),
        mesh=_sc_mesh(),
        compiler_params=_SC_PARAMS,
        scratch_types=[
            pltpu.VMEM((N,), f32), pltpu.VMEM((N,), f32), pltpu.VMEM((N,), f32),
            pltpu.VMEM((CHUNK,), i32), pltpu.VMEM((CHUNK,), i32),
            pltpu.VMEM((CHUNK,), f32), pltpu.VMEM((N,), f32),
        ],
    )
    return fn(row, col, px, py, pz)


ECHUNK = 40
ECPW = E // (ECHUNK * NW)
NSD = 3
NSI = 4
NUNROLL = 12


def _sc_edge_body(a_h, b_h, row_h, col_h, dist_h, w1c_h, accp_h,
                  acc, abuf, bbuf, rowv, colv, distv, w1cv,
                  sem_i, sem_g, sem_s):
    c_ax = lax.axis_index("c")
    s_ax = lax.axis_index("s")
    w = c_ax * NS + s_ax
    pltpu.sync_copy(w1c_h, w1cv)

    def zero_row(i, _):
        for k in range(H // 16):
            abuf[0][i, pl.ds(16 * k, 16)] = jnp.zeros((16,), f32)
        return 0
    lax.fori_loop(0, ECHUNK, zero_row, 0)
    for t in range(NPT // ECHUNK + 1):
        nrows = min(ECHUNK, NPT - t * ECHUNK)
        pltpu.sync_copy(abuf[0].at[pl.ds(0, nrows)],
                        acc.at[pl.ds(s_ax * NPT + t * ECHUNK, nrows)])

    @pl.when(s_ax == NS - 1)
    def _():
        pltpu.sync_copy(abuf[0].at[pl.ds(0, NTAIL)],
                        acc.at[pl.ds(NPT * NS, NTAIL)])
    plsc.subcore_barrier()

    base = w * ECPW

    def issue_idx(ch, sl):
        off = (base + ch) * ECHUNK
        pltpu.async_copy(row_h.at[pl.ds(off, ECHUNK)], rowv[sl], sem_i[sl])
        pltpu.async_copy(col_h.at[pl.ds(off, ECHUNK)], colv[sl], sem_i[sl])
        pltpu.async_copy(dist_h.at[pl.ds(off, ECHUNK)], distv[sl], sem_i[sl])

    def wait_idx(sl):
        pltpu.make_async_copy(row_h.at[pl.ds(0, ECHUNK)], rowv[sl],
                              sem_i[sl]).wait()
        pltpu.make_async_copy(col_h.at[pl.ds(0, ECHUNK)], colv[sl],
                              sem_i[sl]).wait()
        pltpu.make_async_copy(dist_h.at[pl.ds(0, ECHUNK)], distv[sl],
                              sem_i[sl]).wait()

    def issue_gather(dsl, isl):
        pltpu.async_copy(a_h.at[colv[isl]], abuf[dsl], sem_g[dsl])
        pltpu.async_copy(b_h.at[rowv[isl]], bbuf[dsl], sem_g[dsl])

    def wait_gather(dsl, isl):
        pltpu.make_async_copy(a_h.at[colv[isl]], abuf[dsl],
                              sem_g[dsl]).wait()
        pltpu.make_async_copy(b_h.at[rowv[isl]], bbuf[dsl],
                              sem_g[dsl]).wait()

    def compute(dsl, isl):
        def erow(e, _):
            d16 = plsc.load_gather(distv[isl], [jnp.full((16,), e, dtype=i32)])
            for k in range(H // 16):
                slc = pl.ds(16 * k, 16)
                v = abuf[dsl][e, slc] + bbuf[dsl][e, slc] + d16 * w1cv[slc]
                sg = 1.0 / (1.0 + jnp.exp(-v))
                abuf[dsl][e, slc] = v * sg
            return 0
        lax.fori_loop(0, ECHUNK, erow, 0)

    issue_idx(0, 0)
    issue_idx(1, 1)
    wait_idx(0)
    issue_gather(0, 0)

    def block(i, _):
        for u in range(NUNROLL):
            c = i * NUNROLL + u
            d0 = u % NSD
            d2 = (u + 1) % NSD
            i0 = u % NSI
            i1 = (u + 1) % NSI
            i2 = (u + 2) % NSI

            @pl.when(jnp.logical_and(c >= 2, c < ECPW + 2))
            def _():
                pltpu.make_async_copy(abuf[d2], acc.at[colv[i2]],
                                      sem_s[d2]).wait()

            @pl.when(c + 2 < ECPW)
            def _():
                issue_idx(c + 2, i2)

            @pl.when(c + 1 < ECPW)
            def _():
                wait_idx(i1)
                issue_gather(d2, i1)

            @pl.when(c < ECPW)
            def _():
                wait_gather(d0, i0)
                compute(d0, i0)
                pltpu.async_copy(abuf[d0], acc.at[colv[i0]], sem_s[d0],
                                 add=True)
        return 0
    lax.fori_loop(0, (ECPW + 2 + NUNROLL) // NUNROLL, block, 0)

    plsc.subcore_barrier()
    pltpu.sync_copy(acc.at[pl.ds(s_ax * NPT, NPT)],
                    accp_h.at[c_ax, pl.ds(s_ax * NPT, NPT)])

    @pl.when(s_ax == NS - 1)
    def _():
        pltpu.sync_copy(acc.at[pl.ds(NPT * NS, NTAIL)],
                        accp_h.at[c_ax, pl.ds(NPT * NS, NTAIL)])


def _sc_edge(a_tab, b_tab, row, col, dist, w1c):
    fn = pl.kernel(
        _sc_edge_body,
        out_type=jax.ShapeDtypeStruct((NC, N, H), f32),
        mesh=_sc_mesh(),
        compiler_params=_SC_PARAMS,
        scratch_types=[
            pltpu.VMEM_SHARED((N, H), f32),
            [pltpu.VMEM((ECHUNK, H), f32) for _ in range(NSD)],
            [pltpu.VMEM((ECHUNK, H), f32) for _ in range(NSD)],
            [pltpu.VMEM((ECHUNK,), i32) for _ in range(NSI)],
            [pltpu.VMEM((ECHUNK,), i32) for _ in range(NSI)],
            [pltpu.VMEM((ECHUNK,), f32) for _ in range(NSI)],
            pltpu.VMEM((H,), f32),
            [pltpu.SemaphoreType.DMA for _ in range(NSI)],
            [pltpu.SemaphoreType.DMA for _ in range(NSD)],
            [pltpu.SemaphoreType.DMA for _ in range(NSD)],
        ],
    )
    return fn(a_tab, b_tab, row, col, dist, w1c)


def _tc_pre_body(x_r, temb_r, tw1t_r, tb1_r, tw2t_r, tb2_r,
                 w1at_r, b1_r, w1bt_r, h_o, a_o, b_o):
    t = _silu(jnp.dot(temb_r[...], tw1t_r[...], preferred_element_type=f32)
              + tb1_r[...])
    tp = jnp.dot(t, tw2t_r[...], preferred_element_type=f32) + tb2_r[...]
    h = x_r[...] + tp
    h_o[...] = h
    a_o[...] = jnp.dot(h, w1at_r[...], preferred_element_type=f32) + b1_r[...]
    b_o[...] = jnp.dot(h, w1bt_r[...], preferred_element_type=f32)


def _tc_pre(x, time_emb, tw1t, tb1, tw2t, tb2, w1at, b1, w1bt):
    return pl.pallas_call(
        _tc_pre_body,
        out_shape=(jax.ShapeDtypeStruct((N, H), f32),
                   jax.ShapeDtypeStruct((N, H), f32),
                   jax.ShapeDtypeStruct((N, H), f32)),
    )(x, time_emb, tw1t, tb1, tw2t, tb2, w1at, b1, w1bt)


def _tc_dist_body(d2_r, cntp_r, dist_o, inv_o):
    dist_o[...] = jnp.sqrt(d2_r[...] + 1e-12)
    cnt = jnp.sum(cntp_r[...], axis=0, keepdims=True)
    inv_o[...] = 1.0 / jnp.maximum(cnt, 1.0)


def _tc_dist(d2m, cntp):
    return pl.pallas_call(
        _tc_dist_body,
        out_shape=(jax.ShapeDtypeStruct(d2m.shape, f32),
                   jax.ShapeDtypeStruct((1, N), f32)),
    )(d2m, cntp)


def _layer_core(h_r, accp_r, inv_r, ew2t_r, eb2_r, nw1t_r, nb1_r,
                nw2t_r, nb2_r, g_r, b_r):
    h = h_r[...]
    acc = (accp_r[0] + accp_r[1]) * inv_r[...]
    agg = jnp.dot(acc, ew2t_r[...], preferred_element_type=f32) + eb2_r[...]
    t = _silu(jnp.dot(h, nw1t_r[...], preferred_element_type=f32) + nb1_r[...])
    nm = jnp.dot(t, nw2t_r[...], preferred_element_type=f32) + nb2_r[...]
    y = h + nm + agg
    mu = jnp.mean(y, axis=-1, keepdims=True)
    yc = y - mu
    var = jnp.mean(yc * yc, axis=-1, keepdims=True)
    return yc * lax.rsqrt(var + 1e-5) * g_r[...] + b_r[...]


def _tc_layer_ab_body(h_r, accp_r, inv_r, ew2t_r, eb2_r, nw1t_r, nb1_r,
                      nw2t_r, nb2_r, g_r, b_r, w1at_r, b1_r, w1bt_r,
                      h_o, a_o, b_o):
    hn = _layer_core(h_r, accp_r, inv_r, ew2t_r, eb2_r, nw1t_r, nb1_r,
                     nw2t_r, nb2_r, g_r, b_r)
    h_o[...] = hn
    a_o[...] = jnp.dot(hn, w1at_r[...], preferred_element_type=f32) + b1_r[...]
    b_o[...] = jnp.dot(hn, w1bt_r[...], preferred_element_type=f32)


def _tc_layer_final_body(h_r, accp_r, inv_r, ew2t_r, eb2_r, nw1t_r, nb1_r,
                         nw2t_r, nb2_r, g_r, b_r, h_o):
    h_o[...] = _layer_core(h_r, accp_r, inv_r, ew2t_r, eb2_r, nw1t_r, nb1_r,
                           nw2t_r, nb2_r, g_r, b_r)


def _tc_layer_ab(h, accp, inv, ew2t, eb2, nw1t, nb1, nw2t, nb2, g, b,
                 w1at_n, b1_n, w1bt_n):
    return pl.pallas_call(
        _tc_layer_ab_body,
        out_shape=(jax.ShapeDtypeStruct((N, H), f32),
                   jax.ShapeDtypeStruct((N, H), f32),
                   jax.ShapeDtypeStruct((N, H), f32)),
    )(h, accp, inv, ew2t, eb2, nw1t, nb1, nw2t, nb2, g, b,
      w1at_n, b1_n, w1bt_n)


def _tc_layer_final(h, accp, inv, ew2t, eb2, nw1t, nb1, nw2t, nb2, g, b):
    return pl.pallas_call(
        _tc_layer_final_body,
        out_shape=jax.ShapeDtypeStruct((N, H), f32),
    )(h, accp, inv, ew2t, eb2, nw1t, nb1, nw2t, nb2, g, b)


def kernel(x, pos, edge_index, time_emb, t_w1, t_b1, t_w2, t_b2,
           edge_w1, edge_b1, edge_w2, edge_b2, node_w1, node_b1,
           node_w2, node_b2, ln_g, ln_b):
    row = edge_index[0]
    col = edge_index[1]
    px = pos[:, 0]
    py = pos[:, 1]
    pz = pos[:, 2]

    w1at = [edge_w1[l][:, :H].T for l in range(3)]
    w1bt = [edge_w1[l][:, H:2 * H].T for l in range(3)]
    w1c = [edge_w1[l][:, 2 * H] for l in range(3)]
    b1 = [edge_b1[l][None, :] for l in range(3)]
    ew2t = [edge_w2[l].T for l in range(3)]
    eb2 = [edge_b2[l][None, :] for l in range(3)]
    nw1t = [node_w1[l].T for l in range(3)]
    nb1 = [node_b1[l][None, :] for l in range(3)]
    nw2t = [node_w2[l].T for l in range(3)]
    nb2 = [node_b2[l][None, :] for l in range(3)]
    g = [ln_g[l][None, :] for l in range(3)]
    b = [ln_b[l][None, :] for l in range(3)]

    h, a_tab, b_tab = _tc_pre(x, time_emb, t_w1.T, t_b1[None, :], t_w2.T,
                              t_b2[None, :], w1at[0], b1[0], w1bt[0])
    d2, cntp = _sc_d2_cnt(row, col, px, py, pz)
    dist2d, inv1n = _tc_dist(d2.reshape(NCHUNKS, CHUNK),
                             cntp.reshape(NW, N))
    dist = dist2d.reshape(E)
    inv = inv1n.reshape(N, 1)

    for l in range(3):
        accp = _sc_edge(a_tab, b_tab, row, col, dist, w1c[l])
        if l < 2:
            h, a_tab, b_tab = _tc_layer_ab(
                h, accp, inv, ew2t[l], eb2[l], nw1t[l], nb1[l], nw2t[l],
                nb2[l], g[l], b[l], w1at[l + 1], b1[l + 1], w1bt[l + 1])
        else:
            h = _tc_layer_final(
                h, accp, inv, ew2t[l], eb2[l], nw1t[l], nb1[l], nw2t[l],
                nb2[l], g[l], b[l])
    return h, pos
